# baseline (speedup 1.0000x reference)
import jax
import jax.numpy as jnp
import numpy as np
from jax.experimental import pallas as pl
from jax.experimental.pallas import tpu as pltpu

_N = 8192
_B = 1024
_T = _N // _B
_NBLK = _T * (_T + 1) // 2

_RMAP, _CMAP = (np.array(x, dtype=np.int32) for x in zip(
    *[(r, c) for r in range(_T) for c in range(r, _T)]))

_NEG_LOG2E = -1.4426950408889634
_TWO_LN2 = 1.3862943611198906
_LN2 = 0.6931471805599453


def _body(rmap, cmap, pr, tr, pc, tc, out, acc):
    acc[...] = jnp.zeros_like(acc)

    def tile_step(t, _):
        r = rmap[t]
        c = cmap[t]
        pr_t = pr.at[r]
        tr_t = tr.at[r]
        pc_v = pc[c]
        tc_v = tc[c]

        zeros = jnp.zeros((8, _B), jnp.float32)
        m_acc = [zeros, zeros]
        g_acc = [zeros, zeros]
        for k in range(_B // 8):
            pr_s = pr_t[8 * k:8 * k + 8, :]
            tr_s = tr_t[8 * k:8 * k + 8, :]
            d = pr_s - pc_v
            a = jnp.abs(d)
            g = jnp.log2(1.0 + jnp.exp2(a * _NEG_LOG2E))
            x = (a
                 - jnp.where(tr_s > tc_v, d, 0.0)
                 + jnp.where(tr_s < tc_v, d, 0.0))
            m_acc[k % 2] = m_acc[k % 2] + x
            g_acc[k % 2] = g_acc[k % 2] + g
        w = jnp.where(r == c, 0.5, 1.0)
        acc[0:8, :] += w * (m_acc[0] + m_acc[1])
        acc[8:16, :] += w * (g_acc[0] + g_acc[1])
        return _

    jax.lax.fori_loop(0, _NBLK, tile_step, 0)

    def fold8(a8):
        tot = a8[:, 0:128]
        for l in range(1, _B // 128):
            tot = tot + a8[:, 128 * l:128 * (l + 1)]
        return tot

    out[...] = fold8(acc[0:8, :]) + _TWO_LN2 * fold8(acc[8:16, :])


def _acc_sums(p_row, t_row, p_col, t_col):
    return pl.pallas_call(
        _body,
        grid_spec=pltpu.PrefetchScalarGridSpec(
            num_scalar_prefetch=2,
            grid=(1,),
            in_specs=[
                pl.BlockSpec((_T, _B, 1), lambda i, rm, cm: (0, 0, 0)),
                pl.BlockSpec((_T, _B, 1), lambda i, rm, cm: (0, 0, 0)),
                pl.BlockSpec((_T, 1, _B), lambda i, rm, cm: (0, 0, 0)),
                pl.BlockSpec((_T, 1, _B), lambda i, rm, cm: (0, 0, 0)),
            ],
            out_specs=pl.BlockSpec((8, 128), lambda i, rm, cm: (0, 0)),
            scratch_shapes=[pltpu.VMEM((16, _B), jnp.float32)],
        ),
        out_shape=jax.ShapeDtypeStruct((8, 128), jnp.float32),
        compiler_params=pltpu.CompilerParams(
            dimension_semantics=("arbitrary",),
        ),
    )(jnp.asarray(_RMAP), jnp.asarray(_CMAP), p_row, t_row, p_col, t_col)


def kernel(preds, targets):
    n = preds.shape[0]
    p_row = preds.reshape(_T, _B, 1)
    t_row = targets.reshape(_T, _B, 1)
    p_col = preds.reshape(_T, 1, _B)
    t_col = targets.reshape(_T, 1, _B)
    acc = _acc_sums(p_row, t_row, p_col, t_col)
    return (jnp.sum(acc) - n * _LN2) / (n * (n - 1))


# speedup vs baseline: 1.0074x; 1.0074x over previous
import jax
import jax.numpy as jnp
import numpy as np
from jax.experimental import pallas as pl
from jax.experimental.pallas import tpu as pltpu

_N = 8192
_B = 1024
_T = _N // _B
_NBLK = _T * (_T + 1) // 2

_RMAP, _CMAP = (np.array(x, dtype=np.int32) for x in zip(
    *[(r, c) for r in range(_T) for c in range(r, _T)]))

_NEG_LOG2E = -1.4426950408889634
_TWO_LN2 = 1.3862943611198906
_LN2 = 0.6931471805599453


def _body(rmap, cmap, pr, tr, pc, tc, out, acc, pb, tb):
    i = pl.program_id(0)
    r = rmap[i]
    c = cmap[i]

    @pl.when(i == 0)
    def _init():
        acc[...] = jnp.zeros_like(acc)

    @pl.when(r == c)
    def _build():
        for k in range(_B // 8):
            pb[8 * k:8 * k + 8, :] = jnp.broadcast_to(
                pr[8 * k:8 * k + 8, :], (8, _B))
            tb[8 * k:8 * k + 8, :] = jnp.broadcast_to(
                tr[8 * k:8 * k + 8, :], (8, _B))

    pc_v = pc[...]
    tc_v = tc[...]

    zeros = jnp.zeros((8, _B), jnp.float32)
    m_acc = [zeros, zeros]
    g_acc = [zeros, zeros]
    for k in range(_B // 8):
        d = pb[8 * k:8 * k + 8, :] - pc_v
        a = jnp.abs(d)
        g = jnp.log2(1.0 + jnp.exp2(a * _NEG_LOG2E))
        t_s = tb[8 * k:8 * k + 8, :]
        x = (a
             - jnp.where(t_s > tc_v, d, 0.0)
             + jnp.where(t_s < tc_v, d, 0.0))
        m_acc[k % 2] = m_acc[k % 2] + x
        g_acc[k % 2] = g_acc[k % 2] + g
    w = jnp.where(r == c, 0.5, 1.0)
    acc[0:8, :] += w * (m_acc[0] + m_acc[1])
    acc[8:16, :] += w * (g_acc[0] + g_acc[1])

    @pl.when(i == _NBLK - 1)
    def _fold():
        def fold8(a8):
            tot = a8[:, 0:128]
            for l in range(1, _B // 128):
                tot = tot + a8[:, 128 * l:128 * (l + 1)]
            return tot

        out[...] = fold8(acc[0:8, :]) + _TWO_LN2 * fold8(acc[8:16, :])


def _acc_sums(p_row, t_row, p_col, t_col):
    return pl.pallas_call(
        _body,
        grid_spec=pltpu.PrefetchScalarGridSpec(
            num_scalar_prefetch=2,
            grid=(_NBLK,),
            in_specs=[
                pl.BlockSpec((_B, 1), lambda i, rm, cm: (rm[i], 0)),
                pl.BlockSpec((_B, 1), lambda i, rm, cm: (rm[i], 0)),
                pl.BlockSpec((1, _B), lambda i, rm, cm: (0, cm[i])),
                pl.BlockSpec((1, _B), lambda i, rm, cm: (0, cm[i])),
            ],
            out_specs=pl.BlockSpec((8, 128), lambda i, rm, cm: (0, 0)),
            scratch_shapes=[
                pltpu.VMEM((16, _B), jnp.float32),
                pltpu.VMEM((_B, _B), jnp.float32),
                pltpu.VMEM((_B, _B), jnp.float32),
            ],
        ),
        out_shape=jax.ShapeDtypeStruct((8, 128), jnp.float32),
        compiler_params=pltpu.CompilerParams(
            dimension_semantics=("arbitrary",),
        ),
    )(jnp.asarray(_RMAP), jnp.asarray(_CMAP), p_row, t_row, p_col, t_col)


def kernel(preds, targets):
    n = preds.shape[0]
    p_row = preds.reshape(n, 1)
    t_row = targets.reshape(n, 1)
    p_col = preds.reshape(1, n)
    t_col = targets.reshape(1, n)
    acc = _acc_sums(p_row, t_row, p_col, t_col)
    return (jnp.sum(acc) - n * _LN2) / (n * (n - 1))


# speedup vs baseline: 1.0871x; 1.0791x over previous
import jax
import jax.numpy as jnp
import numpy as np
from jax.experimental import pallas as pl
from jax.experimental.pallas import tpu as pltpu

_N = 8192
_B = 1024
_T = _N // _B
_NBLK = _T * (_T + 1) // 2

_RMAP, _CMAP = (np.array(x, dtype=np.int32) for x in zip(
    *[(r, c) for r in range(_T) for c in range(r, _T)]))

_NEG_LOG2E = -1.4426950408889634
_TWO_LN2 = 1.3862943611198906
_LN2 = 0.6931471805599453


def _body(rmap, cmap, pr, tr, pc, tc, out, acc):
    i = pl.program_id(0)
    r = rmap[i]
    c = cmap[i]

    @pl.when(i == 0)
    def _init():
        acc[...] = jnp.zeros_like(acc)

    pc_v = pc[...]
    tc_v = tc[...]

    zeros = jnp.zeros((8, _B), jnp.float32)
    m_acc = [zeros, zeros]
    g_acc = [zeros, zeros]
    for k in range(_B // 8):
        pr_s = pr[8 * k:8 * k + 8, :]
        tr_s = tr[8 * k:8 * k + 8, :]
        d = pr_s - pc_v
        a = jnp.abs(d)
        g = jnp.log2(1.0 + jnp.exp2(a * _NEG_LOG2E))
        x = (a
             - jnp.where(tr_s > tc_v, d, 0.0)
             + jnp.where(tr_s < tc_v, d, 0.0))
        m_acc[k % 2] = m_acc[k % 2] + x
        g_acc[k % 2] = g_acc[k % 2] + g
    w = jnp.where(r == c, 0.5, 1.0)
    acc[0:8, :] += w * (m_acc[0] + m_acc[1])
    acc[8:16, :] += w * (g_acc[0] + g_acc[1])

    @pl.when(i == _NBLK - 1)
    def _fold():
        def fold8(a8):
            tot = a8[:, 0:128]
            for l in range(1, _B // 128):
                tot = tot + a8[:, 128 * l:128 * (l + 1)]
            return tot

        s = fold8(acc[0:8, :]) + _TWO_LN2 * fold8(acc[8:16, :])
        total = jnp.sum(s)
        out[0, 0, 0] = (total - _N * _LN2) / (_N * (_N - 1.0))


def _loss(p_row, t_row, p_col, t_col):
    return pl.pallas_call(
        _body,
        grid_spec=pltpu.PrefetchScalarGridSpec(
            num_scalar_prefetch=2,
            grid=(_NBLK,),
            in_specs=[
                pl.BlockSpec((_B, 1), lambda i, rm, cm: (rm[i], 0)),
                pl.BlockSpec((_B, 1), lambda i, rm, cm: (rm[i], 0)),
                pl.BlockSpec((1, _B), lambda i, rm, cm: (0, cm[i])),
                pl.BlockSpec((1, _B), lambda i, rm, cm: (0, cm[i])),
            ],
            out_specs=pl.BlockSpec((1, 1, 1), lambda i, rm, cm: (0, 0, 0),
                                   memory_space=pltpu.SMEM),
            scratch_shapes=[pltpu.VMEM((16, _B), jnp.float32)],
        ),
        out_shape=jax.ShapeDtypeStruct((1, 1, 1), jnp.float32),
        compiler_params=pltpu.CompilerParams(
            dimension_semantics=("arbitrary",),
        ),
    )(jnp.asarray(_RMAP), jnp.asarray(_CMAP), p_row, t_row, p_col, t_col)


def kernel(preds, targets):
    n = preds.shape[0]
    p_row = preds.reshape(n, 1)
    t_row = targets.reshape(n, 1)
    p_col = preds.reshape(1, n)
    t_col = targets.reshape(1, n)
    res = _loss(p_row, t_row, p_col, t_col)
    return res.reshape(())
